# baseline (speedup 1.0000x reference)
import jax
import jax.numpy as jnp
from jax import lax
from jax.experimental import pallas as pl
from jax.experimental.pallas import tpu as pltpu

NUM_ENTITIES = 15000
NUM_RULES = 50
HIDDEN = 128
BATCH = 32
FLAT = BATCH * NUM_ENTITIES
TILE = 1920
NTILES = FLAT // TILE


def _body(all_r_ref, cnt_ref, rule_emb_ref, rel_emb_ref, g_ref, b_ref,
          W1_ref, b1_ref, W2_ref, b2_ref, out_ref):
    cnt = cnt_ref[...].astype(jnp.float32)
    mask = jnp.sum(cnt, axis=0)
    msg = lax.dot_general(cnt.astype(jnp.bfloat16),
                          rule_emb_ref[...].astype(jnp.bfloat16),
                          (((0,), (0,)), ((), ())),
                          preferred_element_type=jnp.float32)
    mu = jnp.mean(msg, axis=-1, keepdims=True)
    var = jnp.mean((msg - mu) ** 2, axis=-1, keepdims=True)
    msg = (msg - mu) * lax.rsqrt(var + 1e-5)
    msg = msg * g_ref[...] + b_ref[...]
    out = jnp.maximum(msg, 0.0)
    q = all_r_ref[0]
    rel = rel_emb_ref[pl.ds(q, 1), :]
    base = lax.dot_general(rel, W1_ref[pl.ds(HIDDEN, HIDDEN), :],
                           (((1,), (0,)), ((), ())),
                           preferred_element_type=jnp.float32) + b1_ref[...]
    h = lax.dot_general(out.astype(jnp.bfloat16),
                        W1_ref[pl.ds(0, HIDDEN), :].astype(jnp.bfloat16),
                        (((1,), (0,)), ((), ())),
                        preferred_element_type=jnp.float32) + base
    h = jnp.maximum(h, 0.0)
    s = lax.dot_general(h, W2_ref[...], (((1,), (0,)), ((), ())),
                        preferred_element_type=jnp.float32)
    s = s[:, 0] + b2_ref[0]
    out_ref[...] = jnp.where(mask != 0.0, s, 0.0)[None, None, :]


def kernel(all_h, all_r, rule_count, rule_emb, relation_emb, ln_gamma,
           ln_beta, W1, b1, W2, b2, ent_bias):
    rc = rule_count.reshape(NUM_RULES, FLAT)
    score = pl.pallas_call(
        _body,
        grid=(NTILES,),
        in_specs=[
            pl.BlockSpec(memory_space=pltpu.SMEM),
            pl.BlockSpec((NUM_RULES, TILE), lambda i: (0, i)),
            pl.BlockSpec((NUM_RULES, HIDDEN), lambda i: (0, 0)),
            pl.BlockSpec((relation_emb.shape[0], HIDDEN), lambda i: (0, 0)),
            pl.BlockSpec((1, HIDDEN), lambda i: (0, 0)),
            pl.BlockSpec((1, HIDDEN), lambda i: (0, 0)),
            pl.BlockSpec((2 * HIDDEN, HIDDEN), lambda i: (0, 0)),
            pl.BlockSpec((1, HIDDEN), lambda i: (0, 0)),
            pl.BlockSpec((HIDDEN, 1), lambda i: (0, 0)),
            pl.BlockSpec(memory_space=pltpu.SMEM),
        ],
        out_specs=pl.BlockSpec((1, 1, TILE), lambda i: (i, 0, 0)),
        out_shape=jax.ShapeDtypeStruct((NTILES, 1, TILE), jnp.float32),
    )(all_r, rc, rule_emb, relation_emb, ln_gamma.reshape(1, HIDDEN),
      ln_beta.reshape(1, HIDDEN), W1, b1.reshape(1, HIDDEN), W2, b2)
    score = score.reshape(BATCH, NUM_ENTITIES) + ent_bias[None, :]
    mask_out = jnp.ones((BATCH, NUM_ENTITIES), dtype=bool)
    return score, mask_out


# speedup vs baseline: 1.1490x; 1.1490x over previous
import jax
import jax.numpy as jnp
from jax import lax
from jax.experimental import pallas as pl
from jax.experimental.pallas import tpu as pltpu

NUM_ENTITIES = 15000
NUM_RULES = 50
HIDDEN = 128
BATCH = 32
FLAT = BATCH * NUM_ENTITIES
TILE = 1920
NTILES = FLAT // TILE


def _body(all_r_ref, cnt_ref, rule_emb_ref, rel_emb_t_ref, g_ref, b_ref,
          W1_ref, b1_ref, W2_ref, b2_ref, out_ref):
    cnt = cnt_ref[...].astype(jnp.float32)
    mask = jnp.sum(cnt, axis=0, keepdims=True)
    msg = lax.dot_general(rule_emb_ref[...], cnt,
                          (((0,), (0,)), ((), ())),
                          preferred_element_type=jnp.float32)
    mu = jnp.mean(msg, axis=0, keepdims=True)
    var = jnp.mean((msg - mu) ** 2, axis=0, keepdims=True)
    msg = (msg - mu) * lax.rsqrt(var + 1e-5)
    msg = msg * g_ref[...] + b_ref[...]
    out = jnp.maximum(msg, 0.0)
    q = all_r_ref[0]
    rel = rel_emb_t_ref[pl.ds(q, 1), :]
    base = lax.dot_general(W1_ref[pl.ds(HIDDEN, HIDDEN), :], rel,
                           (((0,), (1,)), ((), ())),
                           preferred_element_type=jnp.float32)
    h = lax.dot_general(W1_ref[pl.ds(0, HIDDEN), :], out,
                        (((0,), (0,)), ((), ())),
                        preferred_element_type=jnp.float32)
    h = jnp.maximum(h + base + b1_ref[...], 0.0)
    s = lax.dot_general(W2_ref[...], h, (((0,), (0,)), ((), ())),
                        preferred_element_type=jnp.float32)
    s = s + b2_ref[0]
    out_ref[...] = jnp.where(mask != 0.0, s, 0.0)[None, :, :]


def kernel(all_h, all_r, rule_count, rule_emb, relation_emb, ln_gamma,
           ln_beta, W1, b1, W2, b2, ent_bias):
    rc = rule_count.reshape(NUM_RULES, FLAT)
    nrel = relation_emb.shape[0]
    score = pl.pallas_call(
        _body,
        grid=(NTILES,),
        in_specs=[
            pl.BlockSpec(memory_space=pltpu.SMEM),
            pl.BlockSpec((NUM_RULES, TILE), lambda i: (0, i)),
            pl.BlockSpec((NUM_RULES, HIDDEN), lambda i: (0, 0)),
            pl.BlockSpec((nrel, HIDDEN), lambda i: (0, 0)),
            pl.BlockSpec((HIDDEN, 1), lambda i: (0, 0)),
            pl.BlockSpec((HIDDEN, 1), lambda i: (0, 0)),
            pl.BlockSpec((2 * HIDDEN, HIDDEN), lambda i: (0, 0)),
            pl.BlockSpec((HIDDEN, 1), lambda i: (0, 0)),
            pl.BlockSpec((HIDDEN, 1), lambda i: (0, 0)),
            pl.BlockSpec(memory_space=pltpu.SMEM),
        ],
        out_specs=pl.BlockSpec((1, 1, TILE), lambda i: (i, 0, 0)),
        out_shape=jax.ShapeDtypeStruct((NTILES, 1, TILE), jnp.float32),
    )(all_r, rc, rule_emb, relation_emb, ln_gamma.reshape(HIDDEN, 1),
      ln_beta.reshape(HIDDEN, 1), W1, b1.reshape(HIDDEN, 1), W2, b2)
    score = score.reshape(BATCH, NUM_ENTITIES) + ent_bias[None, :]
    mask_out = jnp.ones((BATCH, NUM_ENTITIES), dtype=bool)
    return score, mask_out


# speedup vs baseline: 1.3682x; 1.1908x over previous
import jax
import jax.numpy as jnp
from jax import lax
from jax.experimental import pallas as pl
from jax.experimental.pallas import tpu as pltpu

NUM_ENTITIES = 15000
NUM_RULES = 50
HIDDEN = 128
BATCH = 32
FLAT = BATCH * NUM_ENTITIES
TILE = 3840
NTILES = FLAT // TILE


def _body(all_r_ref, cnt_ref, rule_emb_ref, rel_emb_t_ref, g_ref, b_ref,
          W1_ref, b1_ref, W2_ref, b2_ref, out_ref):
    cnt = cnt_ref[...].astype(jnp.float32)
    mask = jnp.sum(cnt, axis=0, keepdims=True)
    msg = lax.dot_general(rule_emb_ref[...], cnt,
                          (((0,), (0,)), ((), ())),
                          preferred_element_type=jnp.float32)
    mu = jnp.mean(msg, axis=0, keepdims=True)
    var = jnp.mean((msg - mu) ** 2, axis=0, keepdims=True)
    msg = (msg - mu) * lax.rsqrt(var + 1e-5)
    msg = msg * g_ref[...] + b_ref[...]
    out = jnp.maximum(msg, 0.0)
    q = all_r_ref[0]
    rel = rel_emb_t_ref[pl.ds(q, 1), :]
    base = lax.dot_general(W1_ref[pl.ds(HIDDEN, HIDDEN), :], rel,
                           (((0,), (1,)), ((), ())),
                           preferred_element_type=jnp.float32)
    h = lax.dot_general(W1_ref[pl.ds(0, HIDDEN), :], out,
                        (((0,), (0,)), ((), ())),
                        preferred_element_type=jnp.float32)
    h = jnp.maximum(h + base + b1_ref[...], 0.0)
    s = lax.dot_general(W2_ref[...], h, (((0,), (0,)), ((), ())),
                        preferred_element_type=jnp.float32)
    s = s + b2_ref[0]
    out_ref[...] = jnp.where(mask != 0.0, s, 0.0)[None, :, :]


def kernel(all_h, all_r, rule_count, rule_emb, relation_emb, ln_gamma,
           ln_beta, W1, b1, W2, b2, ent_bias):
    rc = rule_count.reshape(NUM_RULES, FLAT)
    nrel = relation_emb.shape[0]
    score = pl.pallas_call(
        _body,
        grid=(NTILES,),
        in_specs=[
            pl.BlockSpec(memory_space=pltpu.SMEM),
            pl.BlockSpec((NUM_RULES, TILE), lambda i: (0, i)),
            pl.BlockSpec((NUM_RULES, HIDDEN), lambda i: (0, 0)),
            pl.BlockSpec((nrel, HIDDEN), lambda i: (0, 0)),
            pl.BlockSpec((HIDDEN, 1), lambda i: (0, 0)),
            pl.BlockSpec((HIDDEN, 1), lambda i: (0, 0)),
            pl.BlockSpec((2 * HIDDEN, HIDDEN), lambda i: (0, 0)),
            pl.BlockSpec((HIDDEN, 1), lambda i: (0, 0)),
            pl.BlockSpec((HIDDEN, 1), lambda i: (0, 0)),
            pl.BlockSpec(memory_space=pltpu.SMEM),
        ],
        out_specs=pl.BlockSpec((1, 1, TILE), lambda i: (i, 0, 0)),
        out_shape=jax.ShapeDtypeStruct((NTILES, 1, TILE), jnp.float32),
    )(all_r, rc, rule_emb, relation_emb, ln_gamma.reshape(HIDDEN, 1),
      ln_beta.reshape(HIDDEN, 1), W1, b1.reshape(HIDDEN, 1), W2, b2)
    score = score.reshape(BATCH, NUM_ENTITIES) + ent_bias[None, :]
    mask_out = jnp.ones((BATCH, NUM_ENTITIES), dtype=bool)
    return score, mask_out
